# baseline (speedup 1.0000x reference)
import functools

import jax
import jax.numpy as jnp
from jax import lax
from jax.experimental import pallas as pl
from jax.experimental.pallas import tpu as pltpu
from jax.experimental.pallas import tpu_sc as plsc

B = 16384
D = 128
NK = 4
NW = 32
BW = B // NW
NG = BW // 16
CH = 256
NC = BW // CH

_mesh = plsc.VectorSubcoreMesh(core_axis_name="c", subcore_axis_name="s")


@functools.partial(
    pl.kernel,
    out_type=jax.ShapeDtypeStruct((B,), jnp.float32),
    mesh=_mesh,
    compiler_params=pltpu.CompilerParams(needs_layout_passes=False),
    scratch_types=[
        pltpu.VMEM((2, CH, D), jnp.float32),
        pltpu.VMEM((BW,), jnp.int32),
        pltpu.VMEM((NK, D), jnp.float32),
        pltpu.VMEM((256,), jnp.float32),
        pltpu.VMEM((BW,), jnp.float32),
        pltpu.SemaphoreType.DMA,
        pltpu.SemaphoreType.DMA,
    ],
)
def _gamma_map_sc(z_hbm, idx_hbm, g_hbm, out_hbm, z_v, idx_v, g_v, part_v, out_v,
                  sem0, sem1):
    wid = lax.axis_index("s") * 2 + lax.axis_index("c")
    base = wid * BW
    sems = (sem0, sem1)

    handles = [
        pltpu.async_copy(z_hbm.at[pl.ds(base, CH)], z_v.at[0], sems[0])
    ]
    pltpu.sync_copy(idx_hbm.at[pl.ds(base, BW)], idx_v)
    pltpu.sync_copy(g_hbm, g_v)

    iota = lax.iota(jnp.int32, 16)

    def make_group(zbuf, cbase):
        def group(g, carry):
            rowbase = cbase + g * 16
            idx16 = idx_v[pl.ds(rowbase, 16)]
            for r in range(16):
                s = idx16[r]
                prods = []
                for j in range(8):
                    zj = zbuf[g * 16 + r, pl.ds(16 * j, 16)]
                    gs = g_v[s, pl.ds(16 * j, 16)]
                    prods.append(zj * gs)
                while len(prods) > 1:
                    prods = [a + b for a, b in zip(prods[::2], prods[1::2])]
                part_v[pl.ds(16 * r, 16)] = prods[0]
            tots = [plsc.load_gather(part_v, [iota * 16 + c]) for c in range(16)]
            while len(tots) > 1:
                tots = [a + b for a, b in zip(tots[::2], tots[1::2])]
            out_v[pl.ds(rowbase, 16)] = tots[0]
            return carry
        return group

    for c in range(NC):
        if c + 1 < NC:
            handles.append(
                pltpu.async_copy(
                    z_hbm.at[pl.ds(base + (c + 1) * CH, CH)],
                    z_v.at[(c + 1) % 2],
                    sems[(c + 1) % 2],
                )
            )
        handles[c].wait()
        lax.fori_loop(0, CH // 16, make_group(z_v.at[c % 2], c * CH), 0)

    pltpu.sync_copy(out_v, out_hbm.at[pl.ds(base, BW)])


def kernel(z, y, gamma):
    idx = y[:, 0].astype(jnp.int32)
    return _gamma_map_sc(z, idx, gamma)


# speedup vs baseline: 1.0919x; 1.0919x over previous
import functools

import jax
import jax.numpy as jnp
from jax import lax
from jax.experimental import pallas as pl
from jax.experimental.pallas import tpu as pltpu
from jax.experimental.pallas import tpu_sc as plsc

B = 16384
D = 128
NK = 4
NW = 32
BW = B // NW
NG = BW // 16
CH = 256
NC = BW // CH

_mesh = plsc.VectorSubcoreMesh(core_axis_name="c", subcore_axis_name="s")


@functools.partial(
    pl.kernel,
    out_type=jax.ShapeDtypeStruct((B,), jnp.float32),
    mesh=_mesh,
    compiler_params=pltpu.CompilerParams(needs_layout_passes=False),
    scratch_types=[
        pltpu.VMEM((2, CH, D), jnp.float32),
        pltpu.VMEM((BW,), jnp.int32),
        pltpu.VMEM((NK, D), jnp.float32),
        pltpu.VMEM((256,), jnp.float32),
        pltpu.VMEM((BW,), jnp.float32),
        pltpu.SemaphoreType.DMA,
        pltpu.SemaphoreType.DMA,
    ],
)
def _gamma_map_sc(z_hbm, idx_hbm, g_hbm, out_hbm, z_v, idx_v, g_v, part_v, out_v,
                  sem0, sem1):
    wid = lax.axis_index("s") * 2 + lax.axis_index("c")
    base = wid * BW
    sems = (sem0, sem1)

    handles = [
        pltpu.async_copy(z_hbm.at[pl.ds(base, CH)], z_v.at[0], sems[0])
    ]
    pltpu.sync_copy(idx_hbm.at[pl.ds(base, BW)], idx_v)
    pltpu.sync_copy(g_hbm, g_v)

    iota = lax.iota(jnp.int32, 16)
    gt = [[g_v[k, pl.ds(16 * j, 16)] for j in range(8)] for k in range(NK)]

    def make_group(zbuf, cbase):
        def group(g, carry):
            rowbase = cbase + g * 16
            idx16 = idx_v[pl.ds(rowbase, 16)]
            for r in range(16):
                s = idx16[r]
                prods = []
                for j in range(8):
                    zj = zbuf[g * 16 + r, pl.ds(16 * j, 16)]
                    gs = jnp.where(
                        s < 2,
                        jnp.where(s == 0, gt[0][j], gt[1][j]),
                        jnp.where(s == 2, gt[2][j], gt[3][j]),
                    )
                    prods.append(zj * gs)
                while len(prods) > 1:
                    prods = [a + b for a, b in zip(prods[::2], prods[1::2])]
                part_v[pl.ds(16 * r, 16)] = prods[0]
            tots = [plsc.load_gather(part_v, [iota * 16 + c]) for c in range(16)]
            while len(tots) > 1:
                tots = [a + b for a, b in zip(tots[::2], tots[1::2])]
            out_v[pl.ds(rowbase, 16)] = tots[0]
            return carry
        return group

    for c in range(NC):
        if c + 1 < NC:
            handles.append(
                pltpu.async_copy(
                    z_hbm.at[pl.ds(base + (c + 1) * CH, CH)],
                    z_v.at[(c + 1) % 2],
                    sems[(c + 1) % 2],
                )
            )
        handles[c].wait()
        lax.fori_loop(0, CH // 16, make_group(z_v.at[c % 2], c * CH), 0)

    pltpu.sync_copy(out_v, out_hbm.at[pl.ds(base, BW)])


def kernel(z, y, gamma):
    idx = y[:, 0].astype(jnp.int32)
    return _gamma_map_sc(z, idx, gamma)
